# Initial kernel scaffold; baseline (speedup 1.0000x reference)
import jax
import jax.numpy as jnp
from jax.experimental import pallas as pl


def kernel(prompt_ids, embedding_table):
    raise NotImplementedError("write your pallas kernel here")


# speedup vs baseline: 5.7536x; 5.7536x over previous
import functools

import jax
import jax.numpy as jnp
from jax import lax
from jax.experimental import pallas as pl
from jax.experimental.pallas import tpu as pltpu
from jax.experimental.pallas import tpu_sc as plsc

_VOCAB = 100000
_EMBED = 128
_BATCH = 1024
_SEQ = 200
_B = _BATCH * _SEQ
_NC = 2
_NS = 16
_NW = _NC * _NS
_BPW = _B // _NW
_CH = 128
_NCHUNK = _BPW // _CH

_mesh = plsc.VectorSubcoreMesh(
    core_axis_name="c", subcore_axis_name="s", num_cores=_NC, num_subcores=_NS
)


@functools.partial(
    pl.kernel,
    out_type=jax.ShapeDtypeStruct((_B, _EMBED), jnp.float32),
    mesh=_mesh,
    scratch_types=[
        pltpu.VMEM((_BPW,), jnp.int32),
        pltpu.VMEM((_CH, _EMBED), jnp.float32),
        pltpu.SemaphoreType.DMA,
    ],
)
def _gather_rows(idx_hbm, table_hbm, out_hbm, idx_v, rows_v, sem):
    wid = lax.axis_index("s") * _NC + lax.axis_index("c")
    base = wid * _BPW
    pltpu.sync_copy(idx_hbm.at[pl.ds(base, _BPW)], idx_v)

    def body(g, carry):
        off = g * _CH
        pltpu.async_copy(
            table_hbm.at[idx_v.at[pl.ds(off, _CH)]], rows_v, sem
        ).wait()
        pltpu.sync_copy(rows_v, out_hbm.at[pl.ds(base + off, _CH)])
        return carry

    lax.fori_loop(0, _NCHUNK, body, 0)


def kernel(prompt_ids, embedding_table):
    idx = prompt_ids.reshape(-1).astype(jnp.int32)
    out = _gather_rows(idx, embedding_table)
    return out.reshape(_BATCH, _SEQ, _EMBED)


# speedup vs baseline: 7.7321x; 1.3439x over previous
import functools

import jax
import jax.numpy as jnp
from jax import lax
from jax.experimental import pallas as pl
from jax.experimental.pallas import tpu as pltpu
from jax.experimental.pallas import tpu_sc as plsc

_VOCAB = 100000
_EMBED = 128
_BATCH = 1024
_SEQ = 200
_B = _BATCH * _SEQ
_NC = 2
_NS = 16
_NW = _NC * _NS
_BPW = _B // _NW
_CH = 64
_NCHUNK = _BPW // _CH
_NBUF = 4
_NITER = _NCHUNK // _NBUF

_mesh = plsc.VectorSubcoreMesh(
    core_axis_name="c", subcore_axis_name="s", num_cores=_NC, num_subcores=_NS
)


@functools.partial(
    pl.kernel,
    out_type=jax.ShapeDtypeStruct((_B, _EMBED), jnp.float32),
    mesh=_mesh,
    scratch_types=[
        pltpu.VMEM((_BPW,), jnp.int32),
        pltpu.VMEM((_NBUF, _CH, _EMBED), jnp.float32),
        [pltpu.SemaphoreType.DMA] * _NBUF,
        [pltpu.SemaphoreType.DMA] * _NBUF,
    ],
)
def _gather_rows(idx_hbm, table_hbm, out_hbm, idx_v, rows_v, gsems, osems):
    wid = lax.axis_index("s") * _NC + lax.axis_index("c")
    base = wid * _BPW
    pltpu.sync_copy(idx_hbm.at[pl.ds(base, _BPW)], idx_v)

    def gather_start(g, b):
        return pltpu.async_copy(
            table_hbm.at[idx_v.at[pl.ds(g * _CH, _CH)]], rows_v.at[b], gsems[b]
        )

    def out_start(g, b):
        return pltpu.async_copy(
            rows_v.at[b], out_hbm.at[pl.ds(base + g * _CH, _CH)], osems[b]
        )

    def out_drain(b):
        pltpu.make_async_copy(
            rows_v.at[b], out_hbm.at[pl.ds(base, _CH)], osems[b]
        ).wait()

    def ring(j, first):
        gbase = j * _NBUF
        descs = []
        for b in range(_NBUF):
            if not first:
                out_drain(b)
            descs.append(gather_start(gbase + b, b))
        for b in range(_NBUF):
            descs[b].wait()
            out_start(gbase + b, b)

    ring(0, True)
    lax.fori_loop(1, _NITER, lambda j, c: (ring(j, False), c)[1], 0)
    for b in range(_NBUF):
        out_drain(b)


def kernel(prompt_ids, embedding_table):
    idx = prompt_ids.reshape(-1).astype(jnp.int32)
    out = _gather_rows(idx, embedding_table)
    return out.reshape(_BATCH, _SEQ, _EMBED)


# speedup vs baseline: 7.8465x; 1.0148x over previous
import functools

import jax
import jax.numpy as jnp
from jax import lax
from jax.experimental import pallas as pl
from jax.experimental.pallas import tpu as pltpu
from jax.experimental.pallas import tpu_sc as plsc

_VOCAB = 100000
_EMBED = 128
_BATCH = 1024
_SEQ = 200
_B = _BATCH * _SEQ
_NC = 2
_NS = 16
_NW = _NC * _NS
_BPW = _B // _NW
_CH = 80
_NCHUNK = _BPW // _CH
_NBUF = 8
_NITER = _NCHUNK // _NBUF

_mesh = plsc.VectorSubcoreMesh(
    core_axis_name="c", subcore_axis_name="s", num_cores=_NC, num_subcores=_NS
)


@functools.partial(
    pl.kernel,
    out_type=jax.ShapeDtypeStruct((_B, _EMBED), jnp.float32),
    mesh=_mesh,
    scratch_types=[
        pltpu.VMEM((_BPW,), jnp.int32),
        pltpu.VMEM((_NBUF, _CH, _EMBED), jnp.float32),
        [pltpu.SemaphoreType.DMA] * _NBUF,
        [pltpu.SemaphoreType.DMA] * _NBUF,
    ],
)
def _gather_rows(idx_hbm, table_hbm, out_hbm, idx_v, rows_v, gsems, osems):
    wid = lax.axis_index("s") * _NC + lax.axis_index("c")
    base = wid * _BPW
    pltpu.sync_copy(idx_hbm.at[pl.ds(base, _BPW)], idx_v)

    def gather_start(g, b):
        return pltpu.async_copy(
            table_hbm.at[idx_v.at[pl.ds(g * _CH, _CH)]], rows_v.at[b], gsems[b]
        )

    def out_start(g, b):
        return pltpu.async_copy(
            rows_v.at[b], out_hbm.at[pl.ds(base + g * _CH, _CH)], osems[b]
        )

    def out_drain(b):
        pltpu.make_async_copy(
            rows_v.at[b], out_hbm.at[pl.ds(base, _CH)], osems[b]
        ).wait()

    def ring(j, first):
        gbase = j * _NBUF
        descs = []
        for b in range(_NBUF):
            if not first:
                out_drain(b)
            descs.append(gather_start(gbase + b, b))
        for b in range(_NBUF):
            descs[b].wait()
            out_start(gbase + b, b)

    ring(0, True)
    lax.fori_loop(1, _NITER, lambda j, c: (ring(j, False), c)[1], 0)
    for b in range(_NBUF):
        out_drain(b)


def kernel(prompt_ids, embedding_table):
    idx = prompt_ids.reshape(-1).astype(jnp.int32)
    out = _gather_rows(idx, embedding_table)
    return out.reshape(_BATCH, _SEQ, _EMBED)
